# Initial kernel scaffold; baseline (speedup 1.0000x reference)
import jax
import jax.numpy as jnp
from jax.experimental import pallas as pl


def kernel(logits, targets):
    raise NotImplementedError("write your pallas kernel here")


# speedup vs baseline: 5.0353x; 5.0353x over previous
import jax
import jax.numpy as jnp
from jax import lax
from jax.experimental import pallas as pl
from jax.experimental.pallas import tpu as pltpu
from jax.experimental.pallas import tpu_sc as plsc

B = 16384
C = 1000
NWORKERS = 32
ROWS_PER_W = B // NWORKERS
RB = 64
NBLK = ROWS_PER_W // RB
NFULL = 62
TAIL_OFF = C - 16
NEG = float("-inf")


def _sc_topk_body(logits_hbm, targets_hbm, negs_hbm, pos_hbm,
                  buf, tbuf, cand, negsbuf, posbuf):
    lanes = lax.iota(jnp.int32, 16)
    neginf16 = jnp.full((16,), NEG, jnp.float32)
    wid = lax.axis_index("s") * 2 + lax.axis_index("c")
    row0 = wid * ROWS_PER_W
    pltpu.sync_copy(targets_hbm.at[pl.ds(row0, ROWS_PER_W)], tbuf)

    def block(bi, carry):
        base = row0 + bi * RB
        pltpu.sync_copy(logits_hbm.at[pl.ds(base * C, RB * C)], buf)

        for g in range(RB // 16):
            tvec = tbuf[pl.ds(bi * RB + g * 16, 16)]
            idx = (g * 16 + lanes) * C + tvec
            pv = plsc.load_gather(buf, [idx])
            posbuf[pl.ds(g * 16, 16)] = pv
            plsc.store_scatter(buf, [idx], neginf16)

        def rowfn(r, rc):
            rb = r * C
            m = buf[pl.ds(rb, 16)]
            for i in range(1, NFULL):
                m = jnp.maximum(m, buf[pl.ds(rb + 16 * i, 16)])
            tail = buf[pl.ds(rb + TAIL_OFF, 16)]
            tail = jnp.where(lanes >= 8, tail, NEG)
            m = jnp.maximum(m, tail)
            ms, _ = plsc.sort_key_val(m, m)
            tau = jnp.max(jnp.where(lanes == 7, ms, NEG))
            tauv = jnp.full((16,), tau, jnp.float32)

            cur = jnp.zeros((16,), jnp.int32)

            def emit(chunk, cur):
                mask = chunk >= tauv
                pidx = plsc.cumsum(jnp.where(mask, 1, 0).astype(jnp.int32))
                plsc.store_scatter(cand, [cur + pidx - 1], chunk, mask=mask)
                return cur + plsc.all_reduce_population_count(mask)

            for i in range(NFULL):
                cur = emit(buf[pl.ds(rb + 16 * i, 16)], cur)
            cur = emit(tail, cur)
            plsc.store_scatter(cand, [cur + lanes], neginf16)
            n = jnp.max(cur)
            nch = (n + 15) // 16

            def merge(j, t):
                c = cand[pl.ds(j * 16, 16)]
                c_asc, _ = plsc.sort_key_val(c, c)
                tm = jnp.maximum(t, c_asc)
                td, _ = plsc.sort_key_val(tm, tm, descending=True)
                return td

            t16 = lax.fori_loop(0, nch, merge, neginf16)
            negsbuf[pl.ds(r * 16, 16)] = t16
            return rc

        lax.fori_loop(0, RB, rowfn, 0)
        pltpu.sync_copy(negsbuf, negs_hbm.at[pl.ds(base * 16, RB * 16)])
        pltpu.sync_copy(posbuf, pos_hbm.at[pl.ds(base, RB)])
        return carry

    lax.fori_loop(0, NBLK, block, 0)


_sc_topk = pl.kernel(
    _sc_topk_body,
    out_type=(
        jax.ShapeDtypeStruct((B * 16,), jnp.float32),
        jax.ShapeDtypeStruct((B,), jnp.float32),
    ),
    mesh=plsc.VectorSubcoreMesh(core_axis_name="c", subcore_axis_name="s"),
    compiler_params=pltpu.CompilerParams(needs_layout_passes=False),
    scratch_types=[
        pltpu.VMEM((RB * C,), jnp.float32),
        pltpu.VMEM((ROWS_PER_W,), jnp.int32),
        pltpu.VMEM((1024,), jnp.float32),
        pltpu.VMEM((RB * 16,), jnp.float32),
        pltpu.VMEM((RB,), jnp.float32),
    ],
)

TC_BLOCK = 1024
TC_GRID = B // TC_BLOCK


def _tc_loss_body(negs_ref, pos_ref, out_ref):
    i = pl.program_id(0)
    neg = negs_ref[...]
    p = pos_ref[...][:, 0]
    col = lax.broadcasted_iota(jnp.int32, (TC_BLOCK, 16), 1)
    negm = jnp.where(col < 9, neg, NEG)
    m = jnp.maximum(jnp.max(negm, axis=1), p)
    s = jnp.sum(jnp.exp(10.0 * (negm - m[:, None])), axis=1)
    s = s + jnp.exp(10.0 * (p - m))
    part = jnp.sum(jnp.log(s) + 10.0 * (m - p)) * (1.0 / B)
    part2d = jnp.full((1, 1), part, jnp.float32)

    @pl.when(i == 0)
    def _():
        out_ref[...] = part2d

    @pl.when(i > 0)
    def _():
        out_ref[...] += part2d


_tc_loss = pl.pallas_call(
    _tc_loss_body,
    grid=(TC_GRID,),
    in_specs=[
        pl.BlockSpec((TC_BLOCK, 16), lambda i: (i, 0)),
        pl.BlockSpec((TC_BLOCK, 1), lambda i: (i, 0)),
    ],
    out_specs=pl.BlockSpec((1, 1), lambda i: (0, 0)),
    out_shape=jax.ShapeDtypeStruct((1, 1), jnp.float32),
)


@jax.jit
def kernel(logits, targets):
    negs_flat, pos = _sc_topk(logits.reshape(-1), targets.astype(jnp.int32))
    loss = _tc_loss(negs_flat.reshape(B, 16), pos.reshape(B, 1))
    return loss[0, 0]


# speedup vs baseline: 12.0642x; 2.3959x over previous
import jax
import jax.numpy as jnp
from jax import lax
from jax.experimental import pallas as pl
from jax.experimental.pallas import tpu as pltpu
from jax.experimental.pallas import tpu_sc as plsc

B = 16384
C = 1000
NWORKERS = 32
ROWS_PER_W = B // NWORKERS
RB = 64
NBLK = ROWS_PER_W // RB
NFULL = 62
TAIL_OFF = C - 16
NEG = float("-inf")


def _sc_topk_body(logits_hbm, targets_hbm, negs_hbm, pos_hbm,
                  buf, tbuf, negsbuf, posbuf):
    lanes = lax.iota(jnp.int32, 16)
    neginf16 = jnp.full((16,), NEG, jnp.float32)
    wid = lax.axis_index("s") * 2 + lax.axis_index("c")
    row0 = wid * ROWS_PER_W
    pltpu.sync_copy(targets_hbm.at[pl.ds(row0, ROWS_PER_W)], tbuf)

    def block(bi, carry):
        base = row0 + bi * RB
        pltpu.sync_copy(logits_hbm.at[pl.ds(base * C, RB * C)], buf)

        for g in range(RB // 16):
            tvec = tbuf[pl.ds(bi * RB + g * 16, 16)]
            idx = (g * 16 + lanes) * C + tvec
            pv = plsc.load_gather(buf, [idx])
            posbuf[pl.ds(g * 16, 16)] = pv
            plsc.store_scatter(buf, [idx], neginf16)

        def merge2(a, b):
            m = jnp.maximum(a, lax.rev(b, (0,)))
            return jnp.sort(m)

        def rowfn(r, rc):
            rb = r * C
            stack = [None] * 7
            for i in range(NFULL + 1):
                if i < NFULL:
                    c = buf[pl.ds(rb + 16 * i, 16)]
                else:
                    c = jnp.where(lanes >= 8, buf[pl.ds(rb + TAIL_OFF, 16)], NEG)
                cur = jnp.sort(c)
                k = 0
                while stack[k] is not None:
                    cur = merge2(stack[k], cur)
                    stack[k] = None
                    k += 1
                stack[k] = cur
            t16 = None
            for s in stack:
                if s is not None:
                    t16 = s if t16 is None else merge2(t16, s)
            negsbuf[pl.ds(r * 16, 16)] = t16
            return rc

        lax.fori_loop(0, RB, rowfn, 0)
        pltpu.sync_copy(negsbuf, negs_hbm.at[pl.ds(base * 16, RB * 16)])
        pltpu.sync_copy(posbuf, pos_hbm.at[pl.ds(base, RB)])
        return carry

    lax.fori_loop(0, NBLK, block, 0)


_sc_topk = pl.kernel(
    _sc_topk_body,
    out_type=(
        jax.ShapeDtypeStruct((B * 16,), jnp.float32),
        jax.ShapeDtypeStruct((B,), jnp.float32),
    ),
    mesh=plsc.VectorSubcoreMesh(core_axis_name="c", subcore_axis_name="s"),
    compiler_params=pltpu.CompilerParams(needs_layout_passes=False),
    scratch_types=[
        pltpu.VMEM((RB * C,), jnp.float32),
        pltpu.VMEM((ROWS_PER_W,), jnp.int32),
        pltpu.VMEM((RB * 16,), jnp.float32),
        pltpu.VMEM((RB,), jnp.float32),
    ],
)

TC_BLOCK = 1024
TC_GRID = B // TC_BLOCK


def _tc_loss_body(negs_ref, pos_ref, out_ref):
    i = pl.program_id(0)
    neg = negs_ref[...]
    p = pos_ref[...][:, 0]
    col = lax.broadcasted_iota(jnp.int32, (TC_BLOCK, 16), 1)
    negm = jnp.where(col >= 7, neg, NEG)
    m = jnp.maximum(jnp.max(negm, axis=1), p)
    s = jnp.sum(jnp.exp(10.0 * (negm - m[:, None])), axis=1)
    s = s + jnp.exp(10.0 * (p - m))
    part = jnp.sum(jnp.log(s) + 10.0 * (m - p)) * (1.0 / B)
    part2d = jnp.full((1, 1), part, jnp.float32)

    @pl.when(i == 0)
    def _():
        out_ref[...] = part2d

    @pl.when(i > 0)
    def _():
        out_ref[...] += part2d


_tc_loss = pl.pallas_call(
    _tc_loss_body,
    grid=(TC_GRID,),
    in_specs=[
        pl.BlockSpec((TC_BLOCK, 16), lambda i: (i, 0)),
        pl.BlockSpec((TC_BLOCK, 1), lambda i: (i, 0)),
    ],
    out_specs=pl.BlockSpec((1, 1), lambda i: (0, 0)),
    out_shape=jax.ShapeDtypeStruct((1, 1), jnp.float32),
)


@jax.jit
def kernel(logits, targets):
    negs_flat, pos = _sc_topk(logits.reshape(-1), targets.astype(jnp.int32))
    loss = _tc_loss(negs_flat.reshape(B, 16), pos.reshape(B, 1))
    return loss[0, 0]


# speedup vs baseline: 15.6444x; 1.2968x over previous
import jax
import jax.numpy as jnp
from jax import lax
from jax.experimental import pallas as pl
from jax.experimental.pallas import tpu as pltpu
from jax.experimental.pallas import tpu_sc as plsc

B = 16384
C = 1000
NWORKERS = 32
ROWS_PER_W = B // NWORKERS
RB = 64
NBLK = ROWS_PER_W // RB
NFULL = 62
TAIL_OFF = C - 16
NEG = float("-inf")


def _sc_topk_body(logits_hbm, negs_hbm, buf, negsbuf):
    lanes = lax.iota(jnp.int32, 16)
    wid = lax.axis_index("s") * 2 + lax.axis_index("c")
    row0 = wid * ROWS_PER_W

    def merge2(a, b):
        m = jnp.maximum(a, lax.rev(b, (0,)))
        return jnp.sort(m)

    def block(bi, carry):
        base = row0 + bi * RB
        pltpu.sync_copy(logits_hbm.at[pl.ds(base, RB)], buf)

        def rowfn(r, rc):
            stack = [None] * 7
            for i in range(NFULL + 1):
                if i < NFULL:
                    c = buf[r, pl.ds(16 * i, 16)]
                else:
                    c = jnp.where(lanes >= 8, buf[r, pl.ds(TAIL_OFF, 16)], NEG)
                cur = jnp.sort(c)
                k = 0
                while stack[k] is not None:
                    cur = merge2(stack[k], cur)
                    stack[k] = None
                    k += 1
                stack[k] = cur
            t16 = None
            for s in stack:
                if s is not None:
                    t16 = s if t16 is None else merge2(t16, s)
            negsbuf[pl.ds(r * 16, 16)] = t16
            return rc

        lax.fori_loop(0, RB, rowfn, 0)
        pltpu.sync_copy(negsbuf, negs_hbm.at[pl.ds(base * 16, RB * 16)])
        return carry

    lax.fori_loop(0, NBLK, block, 0)


_sc_topk = pl.kernel(
    _sc_topk_body,
    out_type=jax.ShapeDtypeStruct((B * 16,), jnp.float32),
    mesh=plsc.VectorSubcoreMesh(core_axis_name="c", subcore_axis_name="s"),
    compiler_params=pltpu.CompilerParams(needs_layout_passes=False),
    scratch_types=[
        pltpu.VMEM((RB, C), jnp.float32),
        pltpu.VMEM((RB * 16,), jnp.float32),
    ],
)

POS_BLK = 1024
POS_GRID = B // POS_BLK


def _tc_pos_body(logits_ref, tgt_ref, out_ref):
    x = logits_ref[...]
    t = tgt_ref[...]
    col = lax.broadcasted_iota(jnp.int32, (POS_BLK, C), 1)
    out_ref[...] = jnp.max(jnp.where(col == t, x, NEG), axis=1, keepdims=True)


_tc_pos = pl.pallas_call(
    _tc_pos_body,
    grid=(POS_GRID,),
    in_specs=[
        pl.BlockSpec((POS_BLK, C), lambda i: (i, 0)),
        pl.BlockSpec((POS_BLK, 1), lambda i: (i, 0)),
    ],
    out_specs=pl.BlockSpec((POS_BLK, 1), lambda i: (i, 0)),
    out_shape=jax.ShapeDtypeStruct((B, 1), jnp.float32),
)

LB = 2048
LGRID = B // LB


def _tc_loss_body(negs_ref, pos_ref, out_ref):
    i = pl.program_id(0)
    t2 = negs_ref[...]
    p = pos_ref[...][:, 0]
    t16 = t2[:, 0]
    t15 = t2[:, 15]
    t14 = t2[:, 14]
    d9 = t2[:, 6]
    cnt_gt = jnp.sum((t2 > p[:, None]).astype(jnp.float32), axis=1)
    removal = p >= t16
    top1 = jnp.where(removal & (cnt_gt == 0.0), t14, t15)
    m = jnp.maximum(p, top1)
    lane = lax.broadcasted_iota(jnp.int32, (LB, 16), 1)
    e = jnp.where(lane >= 6, jnp.exp(10.0 * (t2 - m[:, None])), 0.0)
    sum10 = jnp.sum(e, axis=1)
    ep = jnp.exp(10.0 * (p - m))
    s = jnp.where(removal & (cnt_gt <= 9.0),
                  sum10,
                  sum10 - jnp.exp(10.0 * (d9 - m)) + ep)
    part = jnp.sum(jnp.log(s) + 10.0 * (m - p)) * (1.0 / B)
    part2d = jnp.full((1, 1), part, jnp.float32)

    @pl.when(i == 0)
    def _():
        out_ref[...] = part2d

    @pl.when(i > 0)
    def _():
        out_ref[...] += part2d


_tc_loss = pl.pallas_call(
    _tc_loss_body,
    grid=(LGRID,),
    in_specs=[
        pl.BlockSpec((LB, 16), lambda i: (i, 0)),
        pl.BlockSpec((LB, 1), lambda i: (i, 0)),
    ],
    out_specs=pl.BlockSpec((1, 1), lambda i: (0, 0)),
    out_shape=jax.ShapeDtypeStruct((1, 1), jnp.float32),
)


@jax.jit
def kernel(logits, targets):
    negs_flat = _sc_topk(logits)
    pos = _tc_pos(logits, targets.astype(jnp.int32).reshape(B, 1))
    loss = _tc_loss(negs_flat.reshape(B, 16), pos)
    return loss[0, 0]


# speedup vs baseline: 18.2399x; 1.1659x over previous
import jax
import jax.numpy as jnp
from jax import lax
from jax.experimental import pallas as pl
from jax.experimental.pallas import tpu as pltpu
from jax.experimental.pallas import tpu_sc as plsc

B = 16384
C = 1000
NWORKERS = 32
ROWS_PER_W = B // NWORKERS
RB = 32
NBLK = ROWS_PER_W // RB
NPAIR = NBLK // 2
NFULL = 62
TAIL_OFF = C - 16
NEG = float("-inf")


def _sc_topk_body(logits_hbm, negs_hbm, buf0, buf1, negsbuf, sem0, sem1):
    lanes = lax.iota(jnp.int32, 16)
    wid = lax.axis_index("s") * 2 + lax.axis_index("c")
    row0 = wid * ROWS_PER_W

    def merge2(a, b):
        m = jnp.maximum(a, lax.rev(b, (0,)))
        return jnp.sort(m)

    def compute_block(buf, base):
        def rowfn(r, rc):
            stack = [None] * 7
            for i in range(NFULL + 1):
                if i < NFULL:
                    c = buf[r, pl.ds(16 * i, 16)]
                else:
                    c = jnp.where(lanes >= 8, buf[r, pl.ds(TAIL_OFF, 16)], NEG)
                cur = jnp.sort(c)
                k = 0
                while stack[k] is not None:
                    cur = merge2(stack[k], cur)
                    stack[k] = None
                    k += 1
                stack[k] = cur
            t16 = None
            for s in stack:
                if s is not None:
                    t16 = s if t16 is None else merge2(t16, s)
            negsbuf[pl.ds(r * 16, 16)] = t16
            return rc

        lax.fori_loop(0, RB, rowfn, 0)
        pltpu.sync_copy(negsbuf, negs_hbm.at[pl.ds(base * 16, RB * 16)])

    def copy_in(bi, buf, sem):
        base = row0 + bi * RB
        return pltpu.async_copy(logits_hbm.at[pl.ds(base, RB)], buf, sem)

    def wait_in(bi, buf, sem):
        base = row0 + bi * RB
        pltpu.make_async_copy(logits_hbm.at[pl.ds(base, RB)], buf, sem).wait()

    copy_in(0, buf0, sem0)

    def pair(pi, carry):
        bi0 = 2 * pi
        wait_in(bi0, buf0, sem0)
        copy_in(bi0 + 1, buf1, sem1)
        compute_block(buf0, row0 + bi0 * RB)
        wait_in(bi0 + 1, buf1, sem1)

        @pl.when(pi + 1 < NPAIR)
        def _():
            copy_in(bi0 + 2, buf0, sem0)

        compute_block(buf1, row0 + (bi0 + 1) * RB)
        return carry

    lax.fori_loop(0, NPAIR, pair, 0)


_sc_topk = pl.kernel(
    _sc_topk_body,
    out_type=jax.ShapeDtypeStruct((B * 16,), jnp.float32),
    mesh=plsc.VectorSubcoreMesh(core_axis_name="c", subcore_axis_name="s"),
    compiler_params=pltpu.CompilerParams(needs_layout_passes=False),
    scratch_types=[
        pltpu.VMEM((RB, C), jnp.float32),
        pltpu.VMEM((RB, C), jnp.float32),
        pltpu.VMEM((RB * 16,), jnp.float32),
        pltpu.SemaphoreType.DMA,
        pltpu.SemaphoreType.DMA,
    ],
)

POS_BLK = 1024
POS_GRID = B // POS_BLK


def _tc_pos_body(logits_ref, tgt_ref, out_ref):
    x = logits_ref[...]
    t = tgt_ref[...]
    col = lax.broadcasted_iota(jnp.int32, (POS_BLK, C), 1)
    out_ref[...] = jnp.max(jnp.where(col == t, x, NEG), axis=1, keepdims=True)


_tc_pos = pl.pallas_call(
    _tc_pos_body,
    grid=(POS_GRID,),
    in_specs=[
        pl.BlockSpec((POS_BLK, C), lambda i: (i, 0)),
        pl.BlockSpec((POS_BLK, 1), lambda i: (i, 0)),
    ],
    out_specs=pl.BlockSpec((POS_BLK, 1), lambda i: (i, 0)),
    out_shape=jax.ShapeDtypeStruct((B, 1), jnp.float32),
)

LB = 2048
LGRID = B // LB


def _tc_loss_body(negs_ref, pos_ref, out_ref):
    i = pl.program_id(0)
    t2 = negs_ref[...]
    p = pos_ref[...][:, 0]
    t16 = t2[:, 0]
    t15 = t2[:, 15]
    t14 = t2[:, 14]
    d9 = t2[:, 6]
    cnt_gt = jnp.sum((t2 > p[:, None]).astype(jnp.float32), axis=1)
    removal = p >= t16
    top1 = jnp.where(removal & (cnt_gt == 0.0), t14, t15)
    m = jnp.maximum(p, top1)
    lane = lax.broadcasted_iota(jnp.int32, (LB, 16), 1)
    e = jnp.where(lane >= 6, jnp.exp(10.0 * (t2 - m[:, None])), 0.0)
    sum10 = jnp.sum(e, axis=1)
    ep = jnp.exp(10.0 * (p - m))
    s = jnp.where(removal & (cnt_gt <= 9.0),
                  sum10,
                  sum10 - jnp.exp(10.0 * (d9 - m)) + ep)
    part = jnp.sum(jnp.log(s) + 10.0 * (m - p)) * (1.0 / B)
    part2d = jnp.full((1, 1), part, jnp.float32)

    @pl.when(i == 0)
    def _():
        out_ref[...] = part2d

    @pl.when(i > 0)
    def _():
        out_ref[...] += part2d


_tc_loss = pl.pallas_call(
    _tc_loss_body,
    grid=(LGRID,),
    in_specs=[
        pl.BlockSpec((LB, 16), lambda i: (i, 0)),
        pl.BlockSpec((LB, 1), lambda i: (i, 0)),
    ],
    out_specs=pl.BlockSpec((1, 1), lambda i: (0, 0)),
    out_shape=jax.ShapeDtypeStruct((1, 1), jnp.float32),
)


@jax.jit
def kernel(logits, targets):
    negs_flat = _sc_topk(logits)
    pos = _tc_pos(logits, targets.astype(jnp.int32).reshape(B, 1))
    loss = _tc_loss(negs_flat.reshape(B, 16), pos)
    return loss[0, 0]
